# Initial kernel scaffold; baseline (speedup 1.0000x reference)
import jax
import jax.numpy as jnp
from jax.experimental import pallas as pl


def kernel(num_layers, x, edge_index, edge_attr, em_w1, em_b1, em_w2, em_b2, mn_w1, mn_b1, mn_w2, mn_b2, en_w1, en_b1, en_w2, en_b2, em_root, em_bias, l1_root, l1_bias, end_root, end_bias, out_w, out_b):
    raise NotImplementedError("write your pallas kernel here")


# speedup vs baseline: 2.7138x; 2.7138x over previous
import functools

import jax
import jax.numpy as jnp
from jax import lax
from jax.experimental import pallas as pl
from jax.experimental.pallas import tpu as pltpu
from jax.experimental.pallas import tpu_sc as plsc

N = 10000
E = 320000
Np = 10240
NC, NS = 2, 16
NW = NC * NS
C = 2048
NCHUNK = 5
EW = C * NCHUNK
Ep = EW * NW
BE = 2560
F = 16


def _mesh():
    return plsc.VectorSubcoreMesh(core_axis_name="c", subcore_axis_name="s",
                                  num_cores=NC, num_subcores=NS)


def _gather_body(h_hbm, src2_hbm, xj_hbm, idx_v, rows_v, sem):
    cid = lax.axis_index("c")
    sid = lax.axis_index("s")
    wid = sid * NC + cid

    @pl.loop(0, NCHUNK)
    def _chunk(ci):
        rowbase = wid * (EW // 128) + ci * (C // 128)
        pltpu.sync_copy(src2_hbm.at[pl.ds(rowbase, C // 128)], idx_v)
        descs = [
            pltpu.async_copy(h_hbm.at[idx_v.at[j]],
                             rows_v.at[pl.ds(j * 128, 128)], sem)
            for j in range(C // 128)
        ]
        for d in descs:
            d.wait()
        ebase = wid * EW + ci * C
        pltpu.sync_copy(rows_v, xj_hbm.at[pl.ds(ebase, C)])


@functools.cache
def _build_gather():
    return pl.kernel(
        _gather_body,
        out_type=jax.ShapeDtypeStruct((Ep, F), jnp.float32),
        mesh=_mesh(),
        scratch_types=[
            pltpu.VMEM((C // 128, 128), jnp.int32),
            pltpu.VMEM((C, F), jnp.float32),
            pltpu.SemaphoreType.DMA,
        ],
        compiler_params=pltpu.CompilerParams(use_tc_tiling_on_sc=False),
    )


def _gather_sc(h, src2):
    return _build_gather()(h, src2)


def _scatter_body(msg_hbm, dst2_hbm, zeros_hbm, out_hbm, idx_v, rows_v,
                  acc_sh, sem):
    cid = lax.axis_index("c")
    sid = lax.axis_index("s")
    wid = sid * NC + cid
    rpt = Np // NS

    pltpu.sync_copy(zeros_hbm.at[pl.ds(sid * rpt, rpt)],
                    acc_sh.at[pl.ds(sid * rpt, rpt)])
    plsc.subcore_barrier()

    @pl.loop(0, NCHUNK)
    def _chunk(ci):
        rowbase = wid * (EW // 128) + ci * (C // 128)
        pltpu.sync_copy(dst2_hbm.at[pl.ds(rowbase, C // 128)], idx_v)
        ebase = wid * EW + ci * C
        pltpu.sync_copy(msg_hbm.at[pl.ds(ebase, C)], rows_v)
        for j in range(C // 128):
            pltpu.sync_copy(rows_v.at[pl.ds(j * 128, 128)],
                            acc_sh.at[idx_v.at[j]], add=True)

    plsc.subcore_barrier()
    pltpu.sync_copy(acc_sh.at[pl.ds(sid * rpt, rpt)],
                    out_hbm.at[pl.ds(cid * Np + sid * rpt, rpt)])


@functools.cache
def _build_scatter():
    return pl.kernel(
        _scatter_body,
        out_type=jax.ShapeDtypeStruct((NC * Np, F), jnp.float32),
        mesh=_mesh(),
        scratch_types=[
            pltpu.VMEM((C // 128, 128), jnp.int32),
            pltpu.VMEM((C, F), jnp.float32),
            pltpu.VMEM_SHARED((Np, F), jnp.float32),
            pltpu.SemaphoreType.DMA,
        ],
        compiler_params=pltpu.CompilerParams(use_tc_tiling_on_sc=False),
    )


def _scatter_sc(msg, dst2, zeros16):
    return _build_scatter()(msg, dst2, zeros16)


def _count_body(dst2_hbm, zeros_hbm, ones_hbm, out_hbm, idx_v, ones_v,
                acc_sh, sem):
    cid = lax.axis_index("c")
    sid = lax.axis_index("s")
    wid = sid * NC + cid
    rpt = Np // NS

    pltpu.sync_copy(ones_hbm, ones_v)
    pltpu.sync_copy(zeros_hbm.at[pl.ds(sid * rpt, rpt)],
                    acc_sh.at[pl.ds(sid * rpt, rpt)])
    plsc.subcore_barrier()

    @pl.loop(0, NCHUNK)
    def _chunk(ci):
        rowbase = wid * (EW // 128) + ci * (C // 128)
        pltpu.sync_copy(dst2_hbm.at[pl.ds(rowbase, C // 128)], idx_v)
        for j in range(C // 128):
            pltpu.sync_copy(ones_v, acc_sh.at[idx_v.at[j]], add=True)

    plsc.subcore_barrier()
    pltpu.sync_copy(acc_sh.at[pl.ds(sid * rpt, rpt)],
                    out_hbm.at[pl.ds(cid * Np + sid * rpt, rpt)])


@functools.cache
def _build_count():
    return pl.kernel(
        _count_body,
        out_type=jax.ShapeDtypeStruct((NC * Np, 8), jnp.float32),
        mesh=_mesh(),
        scratch_types=[
            pltpu.VMEM((C // 128, 128), jnp.int32),
            pltpu.VMEM((128, 8), jnp.float32),
            pltpu.VMEM_SHARED((Np, 8), jnp.float32),
            pltpu.SemaphoreType.DMA,
        ],
        compiler_params=pltpu.CompilerParams(use_tc_tiling_on_sc=False),
    )


def _count_sc(dst2, zeros8, ones8):
    return _build_count()(dst2, zeros8, ones8)


def _msg_body(ea_ref, xj_ref, w1_ref, b1_ref, w2_ref, b2_ref, out_ref):
    a = jnp.maximum(
        jnp.dot(ea_ref[...], w1_ref[...],
                preferred_element_type=jnp.float32) + b1_ref[...], 0.0)
    xj = xj_ref[...]
    acc = jnp.dot(xj, b2_ref[...], preferred_element_type=jnp.float32)
    for h in range(10):
        acc = acc + jnp.dot(a[:, h:h + 1] * xj,
                            w2_ref[h * F:(h + 1) * F, :],
                            preferred_element_type=jnp.float32)
    out_ref[...] = acc


def _msg_tc(ea, xj, w1p, b1p, w2r, b2r):
    return pl.pallas_call(
        _msg_body,
        grid=(Ep // BE,),
        in_specs=[
            pl.BlockSpec((BE, 8), lambda i: (i, 0)),
            pl.BlockSpec((BE, F), lambda i: (i, 0)),
            pl.BlockSpec((8, F), lambda i: (0, 0)),
            pl.BlockSpec((1, F), lambda i: (0, 0)),
            pl.BlockSpec((10 * F, F), lambda i: (0, 0)),
            pl.BlockSpec((F, F), lambda i: (0, 0)),
        ],
        out_specs=pl.BlockSpec((BE, F), lambda i: (i, 0)),
        out_shape=jax.ShapeDtypeStruct((Ep, F), jnp.float32),
    )(ea, xj, w1p, b1p, w2r, b2r)


def _upd_body(sp_ref, cp_ref, h_ref, root_ref, bias_ref, out_ref):
    s = sp_ref[pl.ds(0, Np), :] + sp_ref[pl.ds(Np, Np), :]
    cnt = cp_ref[pl.ds(0, Np), :] + cp_ref[pl.ds(Np, Np), :]
    mean = s / jnp.maximum(cnt[:, 0:1], 1.0)
    out_ref[...] = jnp.maximum(
        mean + jnp.dot(h_ref[...], root_ref[...],
                       preferred_element_type=jnp.float32) + bias_ref[...],
        0.0)


def _upd_tc(spart, cpart, h_prev, rootp, biasp):
    return pl.pallas_call(
        _upd_body,
        in_specs=[
            pl.BlockSpec((NC * Np, F), lambda: (0, 0)),
            pl.BlockSpec((NC * Np, 8), lambda: (0, 0)),
            pl.BlockSpec((Np, F), lambda: (0, 0)),
            pl.BlockSpec((F, F), lambda: (0, 0)),
            pl.BlockSpec((1, F), lambda: (0, 0)),
        ],
        out_specs=pl.BlockSpec((Np, F), lambda: (0, 0)),
        out_shape=jax.ShapeDtypeStruct((Np, F), jnp.float32),
    )(spart, cpart, h_prev, rootp, biasp)


def _upd3_body(sp_ref, cp_ref, h_ref, root_ref, bias_ref, ow_ref, ob_ref,
               out_ref):
    s = sp_ref[pl.ds(0, Np), :] + sp_ref[pl.ds(Np, Np), :]
    cnt = cp_ref[pl.ds(0, Np), :] + cp_ref[pl.ds(Np, Np), :]
    mean = s / jnp.maximum(cnt[:, 0:1], 1.0)
    h3 = jnp.maximum(
        mean + jnp.dot(h_ref[...], root_ref[...],
                       preferred_element_type=jnp.float32) + bias_ref[...],
        0.0)
    out_ref[...] = jnp.dot(h3, ow_ref[...],
                           preferred_element_type=jnp.float32) + ob_ref[...]


def _upd3_tc(spart, cpart, h_prev, rootp, biasp, owp, obp):
    return pl.pallas_call(
        _upd3_body,
        in_specs=[
            pl.BlockSpec((NC * Np, F), lambda: (0, 0)),
            pl.BlockSpec((NC * Np, 8), lambda: (0, 0)),
            pl.BlockSpec((Np, F), lambda: (0, 0)),
            pl.BlockSpec((F, F), lambda: (0, 0)),
            pl.BlockSpec((1, F), lambda: (0, 0)),
            pl.BlockSpec((F, 8), lambda: (0, 0)),
            pl.BlockSpec((1, 8), lambda: (0, 0)),
        ],
        out_specs=pl.BlockSpec((Np, 8), lambda: (0, 0)),
        out_shape=jax.ShapeDtypeStruct((Np, 8), jnp.float32),
    )(spart, cpart, h_prev, rootp, biasp, owp, obp)


def _prep_layer(w1, b1, w2, b2, root, bias, in_c, out_c):
    w1p = jnp.zeros((8, F), jnp.float32).at[:5, :10].set(w1)
    b1p = jnp.zeros((1, F), jnp.float32).at[0, :10].set(b1)
    w2r = w2.reshape(10, in_c, out_c)
    w2r = jnp.pad(w2r, ((0, 0), (0, F - in_c), (0, F - out_c)))
    w2r = w2r.reshape(10 * F, F)
    b2r = jnp.pad(b2.reshape(in_c, out_c),
                  ((0, F - in_c), (0, F - out_c)))
    rootp = jnp.pad(root, ((0, F - in_c), (0, F - out_c)))
    biasp = jnp.pad(bias, (0, F - out_c)).reshape(1, F)
    return w1p, b1p, w2r, b2r, rootp, biasp


def _layer(h, ea, src2, dst2, zeros16, cpart, params, last=False,
           head=None):
    w1p, b1p, w2r, b2r, rootp, biasp = params
    xj = _gather_sc(h, src2)
    msg = _msg_tc(ea, xj, w1p, b1p, w2r, b2r)
    spart = _scatter_sc(msg, dst2, zeros16)
    if last:
        owp, obp = head
        return _upd3_tc(spart, cpart, h, rootp, biasp, owp, obp)
    return _upd_tc(spart, cpart, h, rootp, biasp)


def kernel(num_layers, x, edge_index, edge_attr,
           em_w1, em_b1, em_w2, em_b2,
           mn_w1, mn_b1, mn_w2, mn_b2,
           en_w1, en_b1, en_w2, en_b2,
           em_root, em_bias, l1_root, l1_bias, end_root, end_bias,
           out_w, out_b):
    src = jnp.pad(edge_index[0], (0, Ep - E))
    dst = jnp.pad(edge_index[1], (0, Ep - E), constant_values=N)
    src2 = src.reshape(Ep // 128, 128)
    dst2 = dst.reshape(Ep // 128, 128)
    ea = jnp.pad(edge_attr, ((0, Ep - E), (0, 3)))
    x_pad = jnp.pad(x, ((0, Np - N), (0, F - 10)))
    zeros16 = jnp.zeros((Np, F), jnp.float32)
    zeros8 = jnp.zeros((Np, 8), jnp.float32)
    ones8 = jnp.ones((128, 8), jnp.float32)

    cpart = _count_sc(dst2, zeros8, ones8)

    p1 = _prep_layer(em_w1, em_b1, em_w2, em_b2, em_root, em_bias, 10, F)
    p2 = _prep_layer(mn_w1, mn_b1, mn_w2, mn_b2, l1_root, l1_bias, F, F)
    p3 = _prep_layer(en_w1, en_b1, en_w2, en_b2, end_root, end_bias, F, 10)
    owp = jnp.zeros((F, 8), jnp.float32).at[:10, :1].set(out_w)
    obp = jnp.zeros((1, 8), jnp.float32).at[0, :1].set(out_b)

    h1 = _layer(x_pad, ea, src2, dst2, zeros16, cpart, p1)
    h_mid = _layer(h1, ea, src2, dst2, zeros16, cpart, p2)
    h2 = jnp.where(num_layers == 1, h_mid, h1)
    out = _layer(h2, ea, src2, dst2, zeros16, cpart, p3, last=True,
                 head=(owp, obp))
    return out[:N, :1]


# speedup vs baseline: 7.7642x; 2.8610x over previous
import functools

import jax
import jax.numpy as jnp
from jax import lax
from jax.experimental import pallas as pl
from jax.experimental.pallas import tpu as pltpu
from jax.experimental.pallas import tpu_sc as plsc

N = 10000
E = 320000
Np = 10240
NC, NS = 2, 16
NW = NC * NS
C = 2048
NCHUNK = 5
EW = C * NCHUNK
Ep = EW * NW
F = 16
R8 = Ep // 8
BR = 512
NH = 10


def _mesh():
    return plsc.VectorSubcoreMesh(core_axis_name="c", subcore_axis_name="s",
                                  num_cores=NC, num_subcores=NS)


def _gather_body(h_hbm, src2_hbm, xj_hbm, idx_v, rows_v, sem):
    cid = lax.axis_index("c")
    sid = lax.axis_index("s")
    wid = sid * NC + cid

    @pl.loop(0, NCHUNK)
    def _chunk(ci):
        rowbase = wid * (EW // 128) + ci * (C // 128)
        pltpu.sync_copy(src2_hbm.at[pl.ds(rowbase, C // 128)], idx_v)
        descs = [
            pltpu.async_copy(h_hbm.at[idx_v.at[j]],
                             rows_v.at[pl.ds(j * 128, 128)], sem)
            for j in range(C // 128)
        ]
        for d in descs:
            d.wait()
        ebase = wid * EW + ci * C
        pltpu.sync_copy(rows_v, xj_hbm.at[pl.ds(ebase, C)])


@functools.cache
def _build_gather():
    return pl.kernel(
        _gather_body,
        out_type=jax.ShapeDtypeStruct((Ep, F), jnp.float32),
        mesh=_mesh(),
        scratch_types=[
            pltpu.VMEM((C // 128, 128), jnp.int32),
            pltpu.VMEM((C, F), jnp.float32),
            pltpu.SemaphoreType.DMA,
        ],
        compiler_params=pltpu.CompilerParams(use_tc_tiling_on_sc=False),
    )


def _gather_sc(h, src2):
    return _build_gather()(h, src2)


def _scatter_body(msg_hbm, dst2_hbm, zeros_hbm, out_hbm, idx_v, rows_v,
                  acc_sh, sem):
    cid = lax.axis_index("c")
    sid = lax.axis_index("s")
    wid = sid * NC + cid
    rpt = Np // NS

    pltpu.sync_copy(zeros_hbm.at[pl.ds(sid * rpt, rpt)],
                    acc_sh.at[pl.ds(sid * rpt, rpt)])
    plsc.subcore_barrier()

    @pl.loop(0, NCHUNK)
    def _chunk(ci):
        rowbase = wid * (EW // 128) + ci * (C // 128)
        pltpu.sync_copy(dst2_hbm.at[pl.ds(rowbase, C // 128)], idx_v)
        ebase = wid * EW + ci * C
        pltpu.sync_copy(msg_hbm.at[pl.ds(ebase, C)], rows_v)
        for j in range(C // 128):
            pltpu.sync_copy(rows_v.at[pl.ds(j * 128, 128)],
                            acc_sh.at[idx_v.at[j]], add=True)

    plsc.subcore_barrier()
    pltpu.sync_copy(acc_sh.at[pl.ds(sid * rpt, rpt)],
                    out_hbm.at[pl.ds(cid * Np + sid * rpt, rpt)])


@functools.cache
def _build_scatter():
    return pl.kernel(
        _scatter_body,
        out_type=jax.ShapeDtypeStruct((NC * Np, F), jnp.float32),
        mesh=_mesh(),
        scratch_types=[
            pltpu.VMEM((C // 128, 128), jnp.int32),
            pltpu.VMEM((C, F), jnp.float32),
            pltpu.VMEM_SHARED((Np, F), jnp.float32),
            pltpu.SemaphoreType.DMA,
        ],
        compiler_params=pltpu.CompilerParams(use_tc_tiling_on_sc=False),
    )


def _scatter_sc(msg, dst2, zeros16):
    return _build_scatter()(msg, dst2, zeros16)


def _scatter_cnt_body(msg_hbm, dst2_hbm, zeros_hbm, zeros8_hbm, ones_hbm,
                      out_hbm, cnt_hbm, idx_v, rows_v, ones_v, acc_sh,
                      cacc_sh, sem):
    cid = lax.axis_index("c")
    sid = lax.axis_index("s")
    wid = sid * NC + cid
    rpt = Np // NS

    pltpu.sync_copy(ones_hbm, ones_v)
    pltpu.sync_copy(zeros_hbm.at[pl.ds(sid * rpt, rpt)],
                    acc_sh.at[pl.ds(sid * rpt, rpt)])
    pltpu.sync_copy(zeros8_hbm.at[pl.ds(sid * rpt, rpt)],
                    cacc_sh.at[pl.ds(sid * rpt, rpt)])
    plsc.subcore_barrier()

    @pl.loop(0, NCHUNK)
    def _chunk(ci):
        rowbase = wid * (EW // 128) + ci * (C // 128)
        pltpu.sync_copy(dst2_hbm.at[pl.ds(rowbase, C // 128)], idx_v)
        ebase = wid * EW + ci * C
        pltpu.sync_copy(msg_hbm.at[pl.ds(ebase, C)], rows_v)
        for j in range(C // 128):
            pltpu.sync_copy(rows_v.at[pl.ds(j * 128, 128)],
                            acc_sh.at[idx_v.at[j]], add=True)
            pltpu.sync_copy(ones_v, cacc_sh.at[idx_v.at[j]], add=True)

    plsc.subcore_barrier()
    pltpu.sync_copy(acc_sh.at[pl.ds(sid * rpt, rpt)],
                    out_hbm.at[pl.ds(cid * Np + sid * rpt, rpt)])
    pltpu.sync_copy(cacc_sh.at[pl.ds(sid * rpt, rpt)],
                    cnt_hbm.at[pl.ds(cid * Np + sid * rpt, rpt)])


@functools.cache
def _build_scatter_cnt():
    return pl.kernel(
        _scatter_cnt_body,
        out_type=[jax.ShapeDtypeStruct((NC * Np, F), jnp.float32),
                  jax.ShapeDtypeStruct((NC * Np, 8), jnp.float32)],
        mesh=_mesh(),
        scratch_types=[
            pltpu.VMEM((C // 128, 128), jnp.int32),
            pltpu.VMEM((C, F), jnp.float32),
            pltpu.VMEM((128, 8), jnp.float32),
            pltpu.VMEM_SHARED((Np, F), jnp.float32),
            pltpu.VMEM_SHARED((Np, 8), jnp.float32),
            pltpu.SemaphoreType.DMA,
        ],
        compiler_params=pltpu.CompilerParams(use_tc_tiling_on_sc=False),
    )


def _scatter_cnt_sc(msg, dst2, zeros16, zeros8, ones8):
    return _build_scatter_cnt()(msg, dst2, zeros16, zeros8, ones8)


def _msg_body(eapk_ref, xj_ref, w1c_ref, b1c_ref, bdc_ref, out_ref):
    z = eapk_ref[...]
    xjb = xj_ref[...]
    a = jnp.maximum(
        jnp.dot(z, w1c_ref[...],
                preferred_element_type=jnp.float32) + b1c_ref[...], 0.0)
    y = jnp.dot(xjb, bdc_ref[...], preferred_element_type=jnp.float32)
    acc = y[:, NH * 128:(NH + 1) * 128]
    for h in range(NH):
        acc = acc + a[:, h * 128:(h + 1) * 128] * y[:, h * 128:(h + 1) * 128]
    out_ref[...] = acc


def _msg_tc(eapk, xj128, w1c, b1c, bdc):
    return pl.pallas_call(
        _msg_body,
        grid=(R8 // BR,),
        in_specs=[
            pl.BlockSpec((BR, 64), lambda i: (i, 0)),
            pl.BlockSpec((BR, 128), lambda i: (i, 0)),
            pl.BlockSpec((64, NH * 128), lambda i: (0, 0)),
            pl.BlockSpec((1, NH * 128), lambda i: (0, 0)),
            pl.BlockSpec((128, (NH + 1) * 128), lambda i: (0, 0)),
        ],
        out_specs=pl.BlockSpec((BR, 128), lambda i: (i, 0)),
        out_shape=jax.ShapeDtypeStruct((R8, 128), jnp.float32),
    )(eapk, xj128, w1c, b1c, bdc)


def _upd_body(sp_ref, cp_ref, h_ref, root_ref, bias_ref, out_ref):
    s = sp_ref[pl.ds(0, Np), :] + sp_ref[pl.ds(Np, Np), :]
    cnt = cp_ref[pl.ds(0, Np), :] + cp_ref[pl.ds(Np, Np), :]
    mean = s / jnp.maximum(cnt[:, 0:1], 1.0)
    out_ref[...] = jnp.maximum(
        mean + jnp.dot(h_ref[...], root_ref[...],
                       preferred_element_type=jnp.float32) + bias_ref[...],
        0.0)


def _upd_tc(spart, cpart, h_prev, rootp, biasp):
    return pl.pallas_call(
        _upd_body,
        in_specs=[
            pl.BlockSpec((NC * Np, F), lambda: (0, 0)),
            pl.BlockSpec((NC * Np, 8), lambda: (0, 0)),
            pl.BlockSpec((Np, F), lambda: (0, 0)),
            pl.BlockSpec((F, F), lambda: (0, 0)),
            pl.BlockSpec((1, F), lambda: (0, 0)),
        ],
        out_specs=pl.BlockSpec((Np, F), lambda: (0, 0)),
        out_shape=jax.ShapeDtypeStruct((Np, F), jnp.float32),
    )(spart, cpart, h_prev, rootp, biasp)


def _upd3_body(sp_ref, cp_ref, h_ref, root_ref, bias_ref, ow_ref, ob_ref,
               out_ref):
    s = sp_ref[pl.ds(0, Np), :] + sp_ref[pl.ds(Np, Np), :]
    cnt = cp_ref[pl.ds(0, Np), :] + cp_ref[pl.ds(Np, Np), :]
    mean = s / jnp.maximum(cnt[:, 0:1], 1.0)
    h3 = jnp.maximum(
        mean + jnp.dot(h_ref[...], root_ref[...],
                       preferred_element_type=jnp.float32) + bias_ref[...],
        0.0)
    out_ref[...] = jnp.dot(h3, ow_ref[...],
                           preferred_element_type=jnp.float32) + ob_ref[...]


def _upd3_tc(spart, cpart, h_prev, rootp, biasp, owp, obp):
    return pl.pallas_call(
        _upd3_body,
        in_specs=[
            pl.BlockSpec((NC * Np, F), lambda: (0, 0)),
            pl.BlockSpec((NC * Np, 8), lambda: (0, 0)),
            pl.BlockSpec((Np, F), lambda: (0, 0)),
            pl.BlockSpec((F, F), lambda: (0, 0)),
            pl.BlockSpec((1, F), lambda: (0, 0)),
            pl.BlockSpec((F, 8), lambda: (0, 0)),
            pl.BlockSpec((1, 8), lambda: (0, 0)),
        ],
        out_specs=pl.BlockSpec((Np, 8), lambda: (0, 0)),
        out_shape=jax.ShapeDtypeStruct((Np, 8), jnp.float32),
    )(spart, cpart, h_prev, rootp, biasp, owp, obp)


def _prep_layer(w1, b1, w2, b2, root, bias, in_c, out_c):
    eye8 = jnp.eye(8, dtype=jnp.float32)
    w1p = jnp.zeros((8, NH), jnp.float32).at[:5, :].set(w1)
    w1c = jnp.concatenate(
        [jnp.kron(eye8, w1p[:, h:h + 1] * jnp.ones((1, F), jnp.float32))
         for h in range(NH)], axis=1)
    b1c = jnp.repeat(b1, 128).reshape(1, NH * 128)
    w2r = w2.reshape(NH, in_c, out_c)
    w2r = jnp.pad(w2r, ((0, 0), (0, F - in_c), (0, F - out_c)))
    b2p = jnp.pad(b2.reshape(in_c, out_c), ((0, F - in_c), (0, F - out_c)))
    bdc = jnp.concatenate(
        [jnp.kron(eye8, w2r[h]) for h in range(NH)]
        + [jnp.kron(eye8, b2p)], axis=1)
    rootp = jnp.pad(root, ((0, F - in_c), (0, F - out_c)))
    biasp = jnp.pad(bias, (0, F - out_c)).reshape(1, F)
    return w1c, b1c, bdc, rootp, biasp


def _layer(h, eapk, src2, dst2, zeros16, cpart, params, first_aux=None,
           last=False, head=None):
    w1c, b1c, bdc, rootp, biasp = params
    xj = _gather_sc(h, src2)
    msg128 = _msg_tc(eapk, xj.reshape(R8, 128), w1c, b1c, bdc)
    msg = msg128.reshape(Ep, F)
    if first_aux is not None:
        zeros8, ones8 = first_aux
        spart, cpart = _scatter_cnt_sc(msg, dst2, zeros16, zeros8, ones8)
    else:
        spart = _scatter_sc(msg, dst2, zeros16)
    if last:
        owp, obp = head
        return _upd3_tc(spart, cpart, h, rootp, biasp, owp, obp)
    return _upd_tc(spart, cpart, h, rootp, biasp), cpart


def kernel(num_layers, x, edge_index, edge_attr,
           em_w1, em_b1, em_w2, em_b2,
           mn_w1, mn_b1, mn_w2, mn_b2,
           en_w1, en_b1, en_w2, en_b2,
           em_root, em_bias, l1_root, l1_bias, end_root, end_bias,
           out_w, out_b):
    src = jnp.pad(edge_index[0], (0, Ep - E))
    dst = jnp.pad(edge_index[1], (0, Ep - E), constant_values=N)
    src2 = src.reshape(Ep // 128, 128)
    dst2 = dst.reshape(Ep // 128, 128)
    eapk = jnp.pad(edge_attr, ((0, Ep - E), (0, 3))).reshape(R8, 64)
    x_pad = jnp.pad(x, ((0, Np - N), (0, F - 10)))
    zeros16 = jnp.zeros((Np, F), jnp.float32)
    zeros8 = jnp.zeros((Np, 8), jnp.float32)
    ones8 = jnp.ones((128, 8), jnp.float32)

    p1 = _prep_layer(em_w1, em_b1, em_w2, em_b2, em_root, em_bias, 10, F)
    p2 = _prep_layer(mn_w1, mn_b1, mn_w2, mn_b2, l1_root, l1_bias, F, F)
    p3 = _prep_layer(en_w1, en_b1, en_w2, en_b2, end_root, end_bias, F, 10)
    owp = jnp.zeros((F, 8), jnp.float32).at[:10, :1].set(out_w)
    obp = jnp.zeros((1, 8), jnp.float32).at[0, :1].set(out_b)

    h1, cpart = _layer(x_pad, eapk, src2, dst2, zeros16, None, p1,
                       first_aux=(zeros8, ones8))
    h_mid, _ = _layer(h1, eapk, src2, dst2, zeros16, cpart, p2)
    h2 = jnp.where(num_layers == 1, h_mid, h1)
    out = _layer(h2, eapk, src2, dst2, zeros16, cpart, p3, last=True,
                 head=(owp, obp))
    return out[:N, :1]


# speedup vs baseline: 7.7765x; 1.0016x over previous
import functools

import jax
import jax.numpy as jnp
from jax import lax
from jax.experimental import pallas as pl
from jax.experimental.pallas import tpu as pltpu
from jax.experimental.pallas import tpu_sc as plsc

N = 10000
E = 320000
Np = 10240
NC, NS = 2, 16
NW = NC * NS
C = 2048
NCHUNK = 5
EW = C * NCHUNK
Ep = EW * NW
F = 16
R8 = Ep // 8
BR = 512
NH = 10


def _mesh():
    return plsc.VectorSubcoreMesh(core_axis_name="c", subcore_axis_name="s",
                                  num_cores=NC, num_subcores=NS)


def _gather_body(h_hbm, src2_hbm, xj_hbm, idx_v, rows_v, sem):
    cid = lax.axis_index("c")
    sid = lax.axis_index("s")
    wid = sid * NC + cid

    @pl.loop(0, NCHUNK)
    def _chunk(ci):
        rowbase = wid * (EW // 128) + ci * (C // 128)
        pltpu.sync_copy(src2_hbm.at[pl.ds(rowbase, C // 128)], idx_v)
        descs = [
            pltpu.async_copy(h_hbm.at[idx_v.at[j]],
                             rows_v.at[pl.ds(j * 128, 128)], sem)
            for j in range(C // 128)
        ]
        for d in descs:
            d.wait()
        ebase = wid * EW + ci * C
        pltpu.sync_copy(rows_v, xj_hbm.at[pl.ds(ebase, C)])


@functools.cache
def _build_gather():
    return pl.kernel(
        _gather_body,
        out_type=jax.ShapeDtypeStruct((Ep, F), jnp.float32),
        mesh=_mesh(),
        scratch_types=[
            pltpu.VMEM((C // 128, 128), jnp.int32),
            pltpu.VMEM((C, F), jnp.float32),
            pltpu.SemaphoreType.DMA,
        ],
        compiler_params=pltpu.CompilerParams(use_tc_tiling_on_sc=False),
    )


def _gather_sc(h, src2):
    return _build_gather()(h, src2)


def _scatter_body(msg_hbm, dst2_hbm, zeros_hbm, out_hbm, idx_v, rows_v,
                  acc_sh, sem):
    cid = lax.axis_index("c")
    sid = lax.axis_index("s")
    wid = sid * NC + cid
    rpt = Np // NS

    pltpu.sync_copy(zeros_hbm.at[pl.ds(sid * rpt, rpt)],
                    acc_sh.at[pl.ds(sid * rpt, rpt)])
    plsc.subcore_barrier()

    @pl.loop(0, NCHUNK)
    def _chunk(ci):
        rowbase = wid * (EW // 128) + ci * (C // 128)
        pltpu.sync_copy(dst2_hbm.at[pl.ds(rowbase, C // 128)], idx_v)
        ebase = wid * EW + ci * C
        pltpu.sync_copy(msg_hbm.at[pl.ds(ebase, C)], rows_v)
        for j in range(C // 128):
            pltpu.sync_copy(rows_v.at[pl.ds(j * 128, 128)],
                            acc_sh.at[idx_v.at[j]], add=True)

    plsc.subcore_barrier()
    pltpu.sync_copy(acc_sh.at[pl.ds(sid * rpt, rpt)],
                    out_hbm.at[pl.ds(cid * Np + sid * rpt, rpt)])


@functools.cache
def _build_scatter():
    return pl.kernel(
        _scatter_body,
        out_type=jax.ShapeDtypeStruct((NC * Np, F), jnp.float32),
        mesh=_mesh(),
        scratch_types=[
            pltpu.VMEM((C // 128, 128), jnp.int32),
            pltpu.VMEM((C, F), jnp.float32),
            pltpu.VMEM_SHARED((Np, F), jnp.float32),
            pltpu.SemaphoreType.DMA,
        ],
        compiler_params=pltpu.CompilerParams(use_tc_tiling_on_sc=False),
    )


def _scatter_sc(msg, dst2, zeros16):
    return _build_scatter()(msg, dst2, zeros16)


def _scatter_cnt_body(msg_hbm, dst2_hbm, zeros_hbm, zeros8_hbm, ones_hbm,
                      out_hbm, cnt_hbm, idx_v, rows_v, ones_v, acc_sh,
                      cacc_sh, sem):
    cid = lax.axis_index("c")
    sid = lax.axis_index("s")
    wid = sid * NC + cid
    rpt = Np // NS

    pltpu.sync_copy(ones_hbm, ones_v)
    pltpu.sync_copy(zeros_hbm.at[pl.ds(sid * rpt, rpt)],
                    acc_sh.at[pl.ds(sid * rpt, rpt)])
    pltpu.sync_copy(zeros8_hbm.at[pl.ds(sid * rpt, rpt)],
                    cacc_sh.at[pl.ds(sid * rpt, rpt)])
    plsc.subcore_barrier()

    @pl.loop(0, NCHUNK)
    def _chunk(ci):
        rowbase = wid * (EW // 128) + ci * (C // 128)
        pltpu.sync_copy(dst2_hbm.at[pl.ds(rowbase, C // 128)], idx_v)
        ebase = wid * EW + ci * C
        pltpu.sync_copy(msg_hbm.at[pl.ds(ebase, C)], rows_v)
        for j in range(C // 128):
            pltpu.sync_copy(rows_v.at[pl.ds(j * 128, 128)],
                            acc_sh.at[idx_v.at[j]], add=True)
            pltpu.sync_copy(ones_v, cacc_sh.at[idx_v.at[j]], add=True)

    plsc.subcore_barrier()
    pltpu.sync_copy(acc_sh.at[pl.ds(sid * rpt, rpt)],
                    out_hbm.at[pl.ds(cid * Np + sid * rpt, rpt)])
    pltpu.sync_copy(cacc_sh.at[pl.ds(sid * rpt, rpt)],
                    cnt_hbm.at[pl.ds(cid * Np + sid * rpt, rpt)])


@functools.cache
def _build_scatter_cnt():
    return pl.kernel(
        _scatter_cnt_body,
        out_type=[jax.ShapeDtypeStruct((NC * Np, F), jnp.float32),
                  jax.ShapeDtypeStruct((NC * Np, 8), jnp.float32)],
        mesh=_mesh(),
        scratch_types=[
            pltpu.VMEM((C // 128, 128), jnp.int32),
            pltpu.VMEM((C, F), jnp.float32),
            pltpu.VMEM((128, 8), jnp.float32),
            pltpu.VMEM_SHARED((Np, F), jnp.float32),
            pltpu.VMEM_SHARED((Np, 8), jnp.float32),
            pltpu.SemaphoreType.DMA,
        ],
        compiler_params=pltpu.CompilerParams(use_tc_tiling_on_sc=False),
    )


def _scatter_cnt_sc(msg, dst2, zeros16, zeros8, ones8):
    return _build_scatter_cnt()(msg, dst2, zeros16, zeros8, ones8)


def _msg_body(ea_ref, xj_ref, w1c_ref, b1c_ref, bdc_ref, out_ref):
    z = ea_ref[...]
    xjb = xj_ref[...]
    a = jnp.maximum(
        jnp.dot(z, w1c_ref[...],
                preferred_element_type=jnp.float32) + b1c_ref[...], 0.0)
    y = jnp.dot(xjb, bdc_ref[...], preferred_element_type=jnp.float32)
    acc = y[:, NH * 128:(NH + 1) * 128]
    for h in range(NH):
        acc = acc + a[:, h * 128:(h + 1) * 128] * y[:, h * 128:(h + 1) * 128]
    out_ref[...] = acc


def _msg_tc(ea, xj128, w1c, b1c, bdc):
    return pl.pallas_call(
        _msg_body,
        grid=(R8 // BR,),
        in_specs=[
            pl.BlockSpec((BR, 40), lambda i: (i, 0)),
            pl.BlockSpec((BR, 128), lambda i: (i, 0)),
            pl.BlockSpec((40, NH * 128), lambda i: (0, 0)),
            pl.BlockSpec((1, NH * 128), lambda i: (0, 0)),
            pl.BlockSpec((128, (NH + 1) * 128), lambda i: (0, 0)),
        ],
        out_specs=pl.BlockSpec((BR, 128), lambda i: (i, 0)),
        out_shape=jax.ShapeDtypeStruct((R8, 128), jnp.float32),
    )(ea, xj128, w1c, b1c, bdc)


def _upd_body(sp_ref, cp_ref, h_ref, root_ref, bias_ref, out_ref):
    s = sp_ref[pl.ds(0, Np), :] + sp_ref[pl.ds(Np, Np), :]
    cnt = cp_ref[pl.ds(0, Np), :] + cp_ref[pl.ds(Np, Np), :]
    mean = s / jnp.maximum(cnt[:, 0:1], 1.0)
    out_ref[...] = jnp.maximum(
        mean + jnp.dot(h_ref[...], root_ref[...],
                       preferred_element_type=jnp.float32) + bias_ref[...],
        0.0)


def _upd_tc(spart, cpart, h_prev, rootp, biasp):
    return pl.pallas_call(
        _upd_body,
        in_specs=[
            pl.BlockSpec((NC * Np, F), lambda: (0, 0)),
            pl.BlockSpec((NC * Np, 8), lambda: (0, 0)),
            pl.BlockSpec((Np, F), lambda: (0, 0)),
            pl.BlockSpec((F, F), lambda: (0, 0)),
            pl.BlockSpec((1, F), lambda: (0, 0)),
        ],
        out_specs=pl.BlockSpec((Np, F), lambda: (0, 0)),
        out_shape=jax.ShapeDtypeStruct((Np, F), jnp.float32),
    )(spart, cpart, h_prev, rootp, biasp)


def _upd3_body(sp_ref, cp_ref, h_ref, root_ref, bias_ref, ow_ref, ob_ref,
               out_ref):
    s = sp_ref[pl.ds(0, Np), :] + sp_ref[pl.ds(Np, Np), :]
    cnt = cp_ref[pl.ds(0, Np), :] + cp_ref[pl.ds(Np, Np), :]
    mean = s / jnp.maximum(cnt[:, 0:1], 1.0)
    h3 = jnp.maximum(
        mean + jnp.dot(h_ref[...], root_ref[...],
                       preferred_element_type=jnp.float32) + bias_ref[...],
        0.0)
    out_ref[...] = jnp.dot(h3, ow_ref[...],
                           preferred_element_type=jnp.float32) + ob_ref[...]


def _upd3_tc(spart, cpart, h_prev, rootp, biasp, owp, obp):
    return pl.pallas_call(
        _upd3_body,
        in_specs=[
            pl.BlockSpec((NC * Np, F), lambda: (0, 0)),
            pl.BlockSpec((NC * Np, 8), lambda: (0, 0)),
            pl.BlockSpec((Np, F), lambda: (0, 0)),
            pl.BlockSpec((F, F), lambda: (0, 0)),
            pl.BlockSpec((1, F), lambda: (0, 0)),
            pl.BlockSpec((F, 8), lambda: (0, 0)),
            pl.BlockSpec((1, 8), lambda: (0, 0)),
        ],
        out_specs=pl.BlockSpec((Np, 8), lambda: (0, 0)),
        out_shape=jax.ShapeDtypeStruct((Np, 8), jnp.float32),
    )(spart, cpart, h_prev, rootp, biasp, owp, obp)


def _prep_layer(w1, b1, w2, b2, root, bias, in_c, out_c):
    eye8 = jnp.eye(8, dtype=jnp.float32)
    w1c = jnp.concatenate(
        [jnp.kron(eye8, w1[:, h:h + 1] * jnp.ones((1, F), jnp.float32))
         for h in range(NH)], axis=1)
    b1c = jnp.repeat(b1, 128).reshape(1, NH * 128)
    w2r = w2.reshape(NH, in_c, out_c)
    w2r = jnp.pad(w2r, ((0, 0), (0, F - in_c), (0, F - out_c)))
    b2p = jnp.pad(b2.reshape(in_c, out_c), ((0, F - in_c), (0, F - out_c)))
    bdc = jnp.concatenate(
        [jnp.kron(eye8, w2r[h]) for h in range(NH)]
        + [jnp.kron(eye8, b2p)], axis=1)
    rootp = jnp.pad(root, ((0, F - in_c), (0, F - out_c)))
    biasp = jnp.pad(bias, (0, F - out_c)).reshape(1, F)
    return w1c, b1c, bdc, rootp, biasp


def _layer(h, ea, src2, dst2, zeros16, cpart, params, first_aux=None,
           last=False, head=None):
    w1c, b1c, bdc, rootp, biasp = params
    xj = _gather_sc(h, src2)
    msg128 = _msg_tc(ea, xj.reshape(R8, 128), w1c, b1c, bdc)
    msg = msg128.reshape(Ep, F)
    if first_aux is not None:
        zeros8, ones8 = first_aux
        spart, cpart = _scatter_cnt_sc(msg, dst2, zeros16, zeros8, ones8)
    else:
        spart = _scatter_sc(msg, dst2, zeros16)
    if last:
        owp, obp = head
        return _upd3_tc(spart, cpart, h, rootp, biasp, owp, obp)
    return _upd_tc(spart, cpart, h, rootp, biasp), cpart


def kernel(num_layers, x, edge_index, edge_attr,
           em_w1, em_b1, em_w2, em_b2,
           mn_w1, mn_b1, mn_w2, mn_b2,
           en_w1, en_b1, en_w2, en_b2,
           em_root, em_bias, l1_root, l1_bias, end_root, end_bias,
           out_w, out_b):
    src = jnp.pad(edge_index[0], (0, Ep - E))
    dst = jnp.pad(edge_index[1], (0, Ep - E), constant_values=N)
    src2 = src.reshape(Ep // 128, 128)
    dst2 = dst.reshape(Ep // 128, 128)
    ea8 = jnp.pad(edge_attr, ((0, Ep - E), (0, 0))).reshape(R8, 40)
    x_pad = jnp.pad(x, ((0, Np - N), (0, F - 10)))
    zeros16 = jnp.zeros((Np, F), jnp.float32)
    zeros8 = jnp.zeros((Np, 8), jnp.float32)
    ones8 = jnp.ones((128, 8), jnp.float32)

    p1 = _prep_layer(em_w1, em_b1, em_w2, em_b2, em_root, em_bias, 10, F)
    p2 = _prep_layer(mn_w1, mn_b1, mn_w2, mn_b2, l1_root, l1_bias, F, F)
    p3 = _prep_layer(en_w1, en_b1, en_w2, en_b2, end_root, end_bias, F, 10)
    owp = jnp.zeros((F, 8), jnp.float32).at[:10, :1].set(out_w)
    obp = jnp.zeros((1, 8), jnp.float32).at[0, :1].set(out_b)

    h1, cpart = _layer(x_pad, ea8, src2, dst2, zeros16, None, p1,
                       first_aux=(zeros8, ones8))
    h_mid, _ = _layer(h1, ea8, src2, dst2, zeros16, cpart, p2)
    h2 = jnp.where(num_layers == 1, h_mid, h1)
    out = _layer(h2, ea8, src2, dst2, zeros16, cpart, p3, last=True,
                 head=(owp, obp))
    return out[:N, :1]


# speedup vs baseline: 7.8806x; 1.0134x over previous
import functools

import jax
import jax.numpy as jnp
from jax import lax
from jax.experimental import pallas as pl
from jax.experimental.pallas import tpu as pltpu
from jax.experimental.pallas import tpu_sc as plsc

N = 10000
E = 320000
Np = 10240
NC, NS = 2, 16
NW = NC * NS
C = 2048
NCHUNK = 5
EW = C * NCHUNK
Ep = EW * NW
F = 16
R8 = Ep // 8
BR = 512
NH = 10


def _mesh():
    return plsc.VectorSubcoreMesh(core_axis_name="c", subcore_axis_name="s",
                                  num_cores=NC, num_subcores=NS)


def _gather_body(h_hbm, src_hbm, xj_hbm, idx_v, rows_v, sem):
    cid = lax.axis_index("c")
    sid = lax.axis_index("s")
    wid = sid * NC + cid

    @pl.loop(0, NCHUNK)
    def _chunk(ci):
        ebase = pl.multiple_of(wid * EW + ci * C, C)

        @pl.when(ebase + C <= E)
        def _full():
            pltpu.sync_copy(src_hbm.at[pl.ds(ebase, C)], idx_v)

        @pl.when(ebase + C > E)
        def _partial():
            for j in range(C // 128):
                @pl.when(ebase + (j + 1) * 128 <= E)
                def _one():
                    off = pl.multiple_of(ebase + j * 128, 128)
                    pltpu.sync_copy(src_hbm.at[pl.ds(off, 128)],
                                    idx_v.at[pl.ds(j * 128, 128)])

        descs = [
            pltpu.async_copy(h_hbm.at[idx_v.at[pl.ds(j * 128, 128)]],
                             rows_v.at[pl.ds(j * 128, 128)], sem)
            for j in range(C // 128)
        ]
        for d in descs:
            d.wait()
        pltpu.sync_copy(rows_v, xj_hbm.at[pl.ds(ebase, C)])


@functools.cache
def _build_gather():
    return pl.kernel(
        _gather_body,
        out_type=jax.ShapeDtypeStruct((Ep, F), jnp.float32),
        mesh=_mesh(),
        scratch_types=[
            pltpu.VMEM((C,), jnp.int32),
            pltpu.VMEM((C, F), jnp.float32),
            pltpu.SemaphoreType.DMA,
        ],
        compiler_params=pltpu.CompilerParams(use_tc_tiling_on_sc=False),
    )


def _gather_sc(h, src1):
    return _build_gather()(h, src1)


def _load_dst_rows(dst_hbm, idx2_v, ebase, sem):
    del sem
    for j in range(C // 128):
        @pl.when(ebase + (j + 1) * 128 <= E)
        def _one():
            off = pl.multiple_of(ebase + j * 128, 128)
            pltpu.sync_copy(dst_hbm.at[pl.ds(off, 128)], idx2_v.at[j])


def _scatter_body(msg_hbm, dst_hbm, zeros_hbm, out_hbm, idx2_v, rows_v,
                  acc_sh, sem):
    cid = lax.axis_index("c")
    sid = lax.axis_index("s")
    wid = sid * NC + cid
    rpt = Np // NS

    pltpu.sync_copy(zeros_hbm.at[pl.ds(sid * rpt, rpt)],
                    acc_sh.at[pl.ds(sid * rpt, rpt)])
    plsc.subcore_barrier()

    @pl.loop(0, NCHUNK)
    def _chunk(ci):
        ebase = pl.multiple_of(wid * EW + ci * C, C)
        _load_dst_rows(dst_hbm, idx2_v, ebase, sem)
        pltpu.sync_copy(msg_hbm.at[pl.ds(ebase, C)], rows_v)
        for j in range(C // 128):
            @pl.when(ebase + (j + 1) * 128 <= E)
            def _add():
                pltpu.sync_copy(rows_v.at[pl.ds(j * 128, 128)],
                                acc_sh.at[idx2_v.at[j]], add=True)

    plsc.subcore_barrier()
    pltpu.sync_copy(acc_sh.at[pl.ds(sid * rpt, rpt)],
                    out_hbm.at[pl.ds(cid * Np + sid * rpt, rpt)])


@functools.cache
def _build_scatter():
    return pl.kernel(
        _scatter_body,
        out_type=jax.ShapeDtypeStruct((NC * Np, F), jnp.float32),
        mesh=_mesh(),
        scratch_types=[
            pltpu.VMEM((C // 128, 128), jnp.int32),
            pltpu.VMEM((C, F), jnp.float32),
            pltpu.VMEM_SHARED((Np, F), jnp.float32),
            pltpu.SemaphoreType.DMA,
        ],
        compiler_params=pltpu.CompilerParams(use_tc_tiling_on_sc=False),
    )


def _scatter_sc(msg, dst1, zeros16):
    return _build_scatter()(msg, dst1, zeros16)


def _scatter_cnt_body(msg_hbm, dst_hbm, zeros_hbm, zeros8_hbm, ones_hbm,
                      out_hbm, cnt_hbm, idx2_v, rows_v, ones_v, acc_sh,
                      cacc_sh, sem):
    cid = lax.axis_index("c")
    sid = lax.axis_index("s")
    wid = sid * NC + cid
    rpt = Np // NS

    pltpu.sync_copy(ones_hbm, ones_v)
    pltpu.sync_copy(zeros_hbm.at[pl.ds(sid * rpt, rpt)],
                    acc_sh.at[pl.ds(sid * rpt, rpt)])
    pltpu.sync_copy(zeros8_hbm.at[pl.ds(sid * rpt, rpt)],
                    cacc_sh.at[pl.ds(sid * rpt, rpt)])
    plsc.subcore_barrier()

    @pl.loop(0, NCHUNK)
    def _chunk(ci):
        ebase = pl.multiple_of(wid * EW + ci * C, C)
        _load_dst_rows(dst_hbm, idx2_v, ebase, sem)
        pltpu.sync_copy(msg_hbm.at[pl.ds(ebase, C)], rows_v)
        for j in range(C // 128):
            @pl.when(ebase + (j + 1) * 128 <= E)
            def _add():
                pltpu.sync_copy(rows_v.at[pl.ds(j * 128, 128)],
                                acc_sh.at[idx2_v.at[j]], add=True)
                pltpu.sync_copy(ones_v, cacc_sh.at[idx2_v.at[j]], add=True)

    plsc.subcore_barrier()
    pltpu.sync_copy(acc_sh.at[pl.ds(sid * rpt, rpt)],
                    out_hbm.at[pl.ds(cid * Np + sid * rpt, rpt)])
    pltpu.sync_copy(cacc_sh.at[pl.ds(sid * rpt, rpt)],
                    cnt_hbm.at[pl.ds(cid * Np + sid * rpt, rpt)])


@functools.cache
def _build_scatter_cnt():
    return pl.kernel(
        _scatter_cnt_body,
        out_type=[jax.ShapeDtypeStruct((NC * Np, F), jnp.float32),
                  jax.ShapeDtypeStruct((NC * Np, 8), jnp.float32)],
        mesh=_mesh(),
        scratch_types=[
            pltpu.VMEM((C // 128, 128), jnp.int32),
            pltpu.VMEM((C, F), jnp.float32),
            pltpu.VMEM((128, 8), jnp.float32),
            pltpu.VMEM_SHARED((Np, F), jnp.float32),
            pltpu.VMEM_SHARED((Np, 8), jnp.float32),
            pltpu.SemaphoreType.DMA,
        ],
        compiler_params=pltpu.CompilerParams(use_tc_tiling_on_sc=False),
    )


def _scatter_cnt_sc(msg, dst1, zeros16, zeros8, ones8):
    return _build_scatter_cnt()(msg, dst1, zeros16, zeros8, ones8)


def _msg_body(ea_ref, xj_ref, w1c_ref, b1c_ref, bdc_ref, out_ref):
    z = ea_ref[...]
    xjb = xj_ref[...]
    a = jnp.maximum(
        jnp.dot(z, w1c_ref[...],
                preferred_element_type=jnp.float32) + b1c_ref[...], 0.0)
    y = jnp.dot(xjb, bdc_ref[...], preferred_element_type=jnp.float32)
    acc = y[:, NH * 128:(NH + 1) * 128]
    for h in range(NH):
        acc = acc + a[:, h * 128:(h + 1) * 128] * y[:, h * 128:(h + 1) * 128]
    out_ref[...] = acc


def _msg_tc(ea, xj128, w1c, b1c, bdc):
    return pl.pallas_call(
        _msg_body,
        grid=(R8 // BR,),
        in_specs=[
            pl.BlockSpec((BR, 40), lambda i: (i, 0)),
            pl.BlockSpec((BR, 128), lambda i: (i, 0)),
            pl.BlockSpec((40, NH * 128), lambda i: (0, 0)),
            pl.BlockSpec((1, NH * 128), lambda i: (0, 0)),
            pl.BlockSpec((128, (NH + 1) * 128), lambda i: (0, 0)),
        ],
        out_specs=pl.BlockSpec((BR, 128), lambda i: (i, 0)),
        out_shape=jax.ShapeDtypeStruct((R8, 128), jnp.float32),
    )(ea, xj128, w1c, b1c, bdc)


def _upd_body(sp_ref, cp_ref, h_ref, root_ref, bias_ref, out_ref):
    s = sp_ref[pl.ds(0, Np), :] + sp_ref[pl.ds(Np, Np), :]
    cnt = cp_ref[pl.ds(0, Np), :] + cp_ref[pl.ds(Np, Np), :]
    mean = s / jnp.maximum(cnt[:, 0:1], 1.0)
    out_ref[...] = jnp.maximum(
        mean + jnp.dot(h_ref[...], root_ref[...],
                       preferred_element_type=jnp.float32) + bias_ref[...],
        0.0)


def _upd_tc(spart, cpart, h_prev, rootp, biasp):
    return pl.pallas_call(
        _upd_body,
        in_specs=[
            pl.BlockSpec((NC * Np, F), lambda: (0, 0)),
            pl.BlockSpec((NC * Np, 8), lambda: (0, 0)),
            pl.BlockSpec((Np, F), lambda: (0, 0)),
            pl.BlockSpec((F, F), lambda: (0, 0)),
            pl.BlockSpec((1, F), lambda: (0, 0)),
        ],
        out_specs=pl.BlockSpec((Np, F), lambda: (0, 0)),
        out_shape=jax.ShapeDtypeStruct((Np, F), jnp.float32),
    )(spart, cpart, h_prev, rootp, biasp)


def _upd3_body(sp_ref, cp_ref, h_ref, root_ref, bias_ref, ow_ref, ob_ref,
               out_ref):
    s = sp_ref[pl.ds(0, Np), :] + sp_ref[pl.ds(Np, Np), :]
    cnt = cp_ref[pl.ds(0, Np), :] + cp_ref[pl.ds(Np, Np), :]
    mean = s / jnp.maximum(cnt[:, 0:1], 1.0)
    h3 = jnp.maximum(
        mean + jnp.dot(h_ref[...], root_ref[...],
                       preferred_element_type=jnp.float32) + bias_ref[...],
        0.0)
    out_ref[...] = jnp.dot(h3, ow_ref[...],
                           preferred_element_type=jnp.float32) + ob_ref[...]


def _upd3_tc(spart, cpart, h_prev, rootp, biasp, owp, obp):
    return pl.pallas_call(
        _upd3_body,
        in_specs=[
            pl.BlockSpec((NC * Np, F), lambda: (0, 0)),
            pl.BlockSpec((NC * Np, 8), lambda: (0, 0)),
            pl.BlockSpec((Np, F), lambda: (0, 0)),
            pl.BlockSpec((F, F), lambda: (0, 0)),
            pl.BlockSpec((1, F), lambda: (0, 0)),
            pl.BlockSpec((F, 8), lambda: (0, 0)),
            pl.BlockSpec((1, 8), lambda: (0, 0)),
        ],
        out_specs=pl.BlockSpec((Np, 8), lambda: (0, 0)),
        out_shape=jax.ShapeDtypeStruct((Np, 8), jnp.float32),
    )(spart, cpart, h_prev, rootp, biasp, owp, obp)


def _prep_layer(w1, b1, w2, b2, root, bias, in_c, out_c):
    eye8 = jnp.eye(8, dtype=jnp.float32)
    w1c = jnp.concatenate(
        [jnp.kron(eye8, w1[:, h:h + 1] * jnp.ones((1, F), jnp.float32))
         for h in range(NH)], axis=1)
    b1c = jnp.repeat(b1, 128).reshape(1, NH * 128)
    w2r = w2.reshape(NH, in_c, out_c)
    w2r = jnp.pad(w2r, ((0, 0), (0, F - in_c), (0, F - out_c)))
    b2p = jnp.pad(b2.reshape(in_c, out_c), ((0, F - in_c), (0, F - out_c)))
    bdc = jnp.concatenate(
        [jnp.kron(eye8, w2r[h]) for h in range(NH)]
        + [jnp.kron(eye8, b2p)], axis=1)
    rootp = jnp.pad(root, ((0, F - in_c), (0, F - out_c)))
    biasp = jnp.pad(bias, (0, F - out_c)).reshape(1, F)
    return w1c, b1c, bdc, rootp, biasp


def _layer(h, ea, src1, dst1, zeros16, cpart, params, first_aux=None,
           last=False, head=None):
    w1c, b1c, bdc, rootp, biasp = params
    xj = _gather_sc(h, src1)
    msg128 = _msg_tc(ea, xj.reshape(R8, 128), w1c, b1c, bdc)
    msg = msg128.reshape(Ep, F)
    if first_aux is not None:
        zeros8, ones8 = first_aux
        spart, cpart = _scatter_cnt_sc(msg, dst1, zeros16, zeros8, ones8)
    else:
        spart = _scatter_sc(msg, dst1, zeros16)
    if last:
        owp, obp = head
        return _upd3_tc(spart, cpart, h, rootp, biasp, owp, obp)
    return _upd_tc(spart, cpart, h, rootp, biasp), cpart


def kernel(num_layers, x, edge_index, edge_attr,
           em_w1, em_b1, em_w2, em_b2,
           mn_w1, mn_b1, mn_w2, mn_b2,
           en_w1, en_b1, en_w2, en_b2,
           em_root, em_bias, l1_root, l1_bias, end_root, end_bias,
           out_w, out_b):
    ea8 = jnp.pad(edge_attr, ((0, Ep - E), (0, 0))).reshape(R8, 40)
    x_pad = jnp.pad(x, ((0, Np - N), (0, F - 10)))
    zeros16 = jnp.zeros((Np, F), jnp.float32)
    zeros8 = jnp.zeros((Np, 8), jnp.float32)
    ones8 = jnp.ones((128, 8), jnp.float32)

    p1 = _prep_layer(em_w1, em_b1, em_w2, em_b2, em_root, em_bias, 10, F)
    p2 = _prep_layer(mn_w1, mn_b1, mn_w2, mn_b2, l1_root, l1_bias, F, F)
    p3 = _prep_layer(en_w1, en_b1, en_w2, en_b2, end_root, end_bias, F, 10)
    owp = jnp.zeros((F, 8), jnp.float32).at[:10, :1].set(out_w)
    obp = jnp.zeros((1, 8), jnp.float32).at[0, :1].set(out_b)

    src1 = edge_index[0]
    dst1 = edge_index[1]
    h1, cpart = _layer(x_pad, ea8, src1, dst1, zeros16, None, p1,
                       first_aux=(zeros8, ones8))
    h_mid, _ = _layer(h1, ea8, src1, dst1, zeros16, cpart, p2)
    h2 = jnp.where(num_layers == 1, h_mid, h1)
    out = _layer(h2, ea8, src1, dst1, zeros16, cpart, p3, last=True,
                 head=(owp, obp))
    return out[:N, :1]


# speedup vs baseline: 9.0153x; 1.1440x over previous
import functools

import jax
import jax.numpy as jnp
from jax import lax
from jax.experimental import pallas as pl
from jax.experimental.pallas import tpu as pltpu
from jax.experimental.pallas import tpu_sc as plsc

N = 10000
E = 320000
Np = 10240
NC, NS = 2, 16
NW = NC * NS
C = 2048
NCHUNK = 5
EW = C * NCHUNK
Ep = EW * NW
F = 16
R8 = Ep // 8
BR = 512
NH = 10


def _mesh():
    return plsc.VectorSubcoreMesh(core_axis_name="c", subcore_axis_name="s",
                                  num_cores=NC, num_subcores=NS)


def _gather_body(h_hbm, src_hbm, xj_hbm, idx_v, rows_v, sem):
    cid = lax.axis_index("c")
    sid = lax.axis_index("s")
    wid = sid * NC + cid

    @pl.loop(0, NCHUNK)
    def _chunk(ci):
        ebase = pl.multiple_of(wid * EW + ci * C, C)

        @pl.when(ebase + C <= E)
        def _full():
            pltpu.sync_copy(src_hbm.at[pl.ds(ebase, C)], idx_v)

        @pl.when(ebase + C > E)
        def _partial():
            for j in range(C // 128):
                @pl.when(ebase + (j + 1) * 128 <= E)
                def _one():
                    off = pl.multiple_of(ebase + j * 128, 128)
                    pltpu.sync_copy(src_hbm.at[pl.ds(off, 128)],
                                    idx_v.at[pl.ds(j * 128, 128)])

        descs = [
            pltpu.async_copy(h_hbm.at[idx_v.at[pl.ds(j * 128, 128)]],
                             rows_v.at[pl.ds(j * 128, 128)], sem)
            for j in range(C // 128)
        ]
        for d in descs:
            d.wait()
        pltpu.sync_copy(rows_v, xj_hbm.at[pl.ds(ebase, C)])


@functools.cache
def _build_gather():
    return pl.kernel(
        _gather_body,
        out_type=jax.ShapeDtypeStruct((Ep, F), jnp.float32),
        mesh=_mesh(),
        scratch_types=[
            pltpu.VMEM((C,), jnp.int32),
            pltpu.VMEM((C, F), jnp.float32),
            pltpu.SemaphoreType.DMA,
        ],
        compiler_params=pltpu.CompilerParams(use_tc_tiling_on_sc=False),
    )


def _gather_sc(h, src1):
    return _build_gather()(h, src1)


def _load_dst_rows(dst2_hbm, idx2_v, ebase):
    rbase = pl.multiple_of(ebase // 128, C // 128)

    @pl.when(ebase + C <= E)
    def _full():
        pltpu.sync_copy(dst2_hbm.at[pl.ds(rbase, C // 128)], idx2_v)

    @pl.when(ebase + C > E)
    def _partial():
        for j in range(C // 128):
            @pl.when(ebase + (j + 1) * 128 <= E)
            def _one():
                pltpu.sync_copy(dst2_hbm.at[pl.ds(rbase + j, 1)],
                                idx2_v.at[pl.ds(j, 1)])


def _scatter_body(msg_hbm, dst_hbm, zeros_hbm, out_hbm, idx2_v, rows_v,
                  acc_sh, sem):
    cid = lax.axis_index("c")
    sid = lax.axis_index("s")
    wid = sid * NC + cid
    rpt = Np // NS

    pltpu.sync_copy(zeros_hbm.at[pl.ds(sid * rpt, rpt)],
                    acc_sh.at[pl.ds(sid * rpt, rpt)])
    plsc.subcore_barrier()

    @pl.loop(0, NCHUNK)
    def _chunk(ci):
        ebase = pl.multiple_of(wid * EW + ci * C, C)
        msg_d = pltpu.async_copy(msg_hbm.at[pl.ds(ebase, C)], rows_v, sem)
        _load_dst_rows(dst_hbm, idx2_v, ebase)
        msg_d.wait()
        for j in range(C // 128):
            @pl.when(ebase + (j + 1) * 128 <= E)
            def _add():
                pltpu.sync_copy(rows_v.at[pl.ds(j * 128, 128)],
                                acc_sh.at[idx2_v.at[j]], add=True)

    plsc.subcore_barrier()
    pltpu.sync_copy(acc_sh.at[pl.ds(sid * rpt, rpt)],
                    out_hbm.at[pl.ds(cid * Np + sid * rpt, rpt)])


@functools.cache
def _build_scatter():
    return pl.kernel(
        _scatter_body,
        out_type=jax.ShapeDtypeStruct((NC * Np, F), jnp.float32),
        mesh=_mesh(),
        scratch_types=[
            pltpu.VMEM((C // 128, 128), jnp.int32),
            pltpu.VMEM((C, F), jnp.float32),
            pltpu.VMEM_SHARED((Np, F), jnp.float32),
            pltpu.SemaphoreType.DMA,
        ],
        compiler_params=pltpu.CompilerParams(use_tc_tiling_on_sc=False),
    )


def _scatter_sc(msg, dst1, zeros16):
    return _build_scatter()(msg, dst1, zeros16)


def _scatter_cnt_body(msg_hbm, dst_hbm, zeros_hbm, zeros8_hbm, ones_hbm,
                      out_hbm, cnt_hbm, idx2_v, rows_v, ones_v, acc_sh,
                      cacc_sh, sem):
    cid = lax.axis_index("c")
    sid = lax.axis_index("s")
    wid = sid * NC + cid
    rpt = Np // NS

    pltpu.sync_copy(ones_hbm, ones_v)
    pltpu.sync_copy(zeros_hbm.at[pl.ds(sid * rpt, rpt)],
                    acc_sh.at[pl.ds(sid * rpt, rpt)])
    pltpu.sync_copy(zeros8_hbm.at[pl.ds(sid * rpt, rpt)],
                    cacc_sh.at[pl.ds(sid * rpt, rpt)])
    plsc.subcore_barrier()

    @pl.loop(0, NCHUNK)
    def _chunk(ci):
        ebase = pl.multiple_of(wid * EW + ci * C, C)
        msg_d = pltpu.async_copy(msg_hbm.at[pl.ds(ebase, C)], rows_v, sem)
        _load_dst_rows(dst_hbm, idx2_v, ebase)
        msg_d.wait()
        for j in range(C // 128):
            @pl.when(ebase + (j + 1) * 128 <= E)
            def _add():
                pltpu.sync_copy(rows_v.at[pl.ds(j * 128, 128)],
                                acc_sh.at[idx2_v.at[j]], add=True)
                pltpu.sync_copy(ones_v, cacc_sh.at[idx2_v.at[j]], add=True)

    plsc.subcore_barrier()
    pltpu.sync_copy(acc_sh.at[pl.ds(sid * rpt, rpt)],
                    out_hbm.at[pl.ds(cid * Np + sid * rpt, rpt)])
    pltpu.sync_copy(cacc_sh.at[pl.ds(sid * rpt, rpt)],
                    cnt_hbm.at[pl.ds(cid * Np + sid * rpt, rpt)])


@functools.cache
def _build_scatter_cnt():
    return pl.kernel(
        _scatter_cnt_body,
        out_type=[jax.ShapeDtypeStruct((NC * Np, F), jnp.float32),
                  jax.ShapeDtypeStruct((NC * Np, 8), jnp.float32)],
        mesh=_mesh(),
        scratch_types=[
            pltpu.VMEM((C // 128, 128), jnp.int32),
            pltpu.VMEM((C, F), jnp.float32),
            pltpu.VMEM((128, 8), jnp.float32),
            pltpu.VMEM_SHARED((Np, F), jnp.float32),
            pltpu.VMEM_SHARED((Np, 8), jnp.float32),
            pltpu.SemaphoreType.DMA,
        ],
        compiler_params=pltpu.CompilerParams(use_tc_tiling_on_sc=False),
    )


def _scatter_cnt_sc(msg, dst1, zeros16, zeros8, ones8):
    return _build_scatter_cnt()(msg, dst1, zeros16, zeros8, ones8)


def _msg_body(ea_ref, xj_ref, w1c_ref, b1c_ref, bdc_ref, out_ref):
    z = ea_ref[...]
    xjb = xj_ref[...]
    a = jnp.maximum(
        jnp.dot(z, w1c_ref[...],
                preferred_element_type=jnp.float32) + b1c_ref[...], 0.0)
    y = jnp.dot(xjb, bdc_ref[...], preferred_element_type=jnp.float32)
    acc = y[:, NH * 128:(NH + 1) * 128]
    for h in range(NH):
        acc = acc + a[:, h * 128:(h + 1) * 128] * y[:, h * 128:(h + 1) * 128]
    out_ref[...] = acc


def _msg_tc(ea, xj128, w1c, b1c, bdc):
    return pl.pallas_call(
        _msg_body,
        grid=(R8 // BR,),
        in_specs=[
            pl.BlockSpec((BR, 40), lambda i: (i, 0)),
            pl.BlockSpec((BR, 128), lambda i: (i, 0)),
            pl.BlockSpec((40, NH * 128), lambda i: (0, 0)),
            pl.BlockSpec((1, NH * 128), lambda i: (0, 0)),
            pl.BlockSpec((128, (NH + 1) * 128), lambda i: (0, 0)),
        ],
        out_specs=pl.BlockSpec((BR, 128), lambda i: (i, 0)),
        out_shape=jax.ShapeDtypeStruct((R8, 128), jnp.float32),
    )(ea, xj128, w1c, b1c, bdc)


def _upd_body(sp_ref, cp_ref, h_ref, root_ref, bias_ref, out_ref):
    s = sp_ref[pl.ds(0, Np), :] + sp_ref[pl.ds(Np, Np), :]
    cnt = cp_ref[pl.ds(0, Np), :] + cp_ref[pl.ds(Np, Np), :]
    mean = s / jnp.maximum(cnt[:, 0:1], 1.0)
    out_ref[...] = jnp.maximum(
        mean + jnp.dot(h_ref[...], root_ref[...],
                       preferred_element_type=jnp.float32) + bias_ref[...],
        0.0)


def _upd_tc(spart, cpart, h_prev, rootp, biasp):
    return pl.pallas_call(
        _upd_body,
        in_specs=[
            pl.BlockSpec((NC * Np, F), lambda: (0, 0)),
            pl.BlockSpec((NC * Np, 8), lambda: (0, 0)),
            pl.BlockSpec((Np, F), lambda: (0, 0)),
            pl.BlockSpec((F, F), lambda: (0, 0)),
            pl.BlockSpec((1, F), lambda: (0, 0)),
        ],
        out_specs=pl.BlockSpec((Np, F), lambda: (0, 0)),
        out_shape=jax.ShapeDtypeStruct((Np, F), jnp.float32),
    )(spart, cpart, h_prev, rootp, biasp)


def _upd3_body(sp_ref, cp_ref, h_ref, root_ref, bias_ref, ow_ref, ob_ref,
               out_ref):
    s = sp_ref[pl.ds(0, Np), :] + sp_ref[pl.ds(Np, Np), :]
    cnt = cp_ref[pl.ds(0, Np), :] + cp_ref[pl.ds(Np, Np), :]
    mean = s / jnp.maximum(cnt[:, 0:1], 1.0)
    h3 = jnp.maximum(
        mean + jnp.dot(h_ref[...], root_ref[...],
                       preferred_element_type=jnp.float32) + bias_ref[...],
        0.0)
    out_ref[...] = jnp.dot(h3, ow_ref[...],
                           preferred_element_type=jnp.float32) + ob_ref[...]


def _upd3_tc(spart, cpart, h_prev, rootp, biasp, owp, obp):
    return pl.pallas_call(
        _upd3_body,
        in_specs=[
            pl.BlockSpec((NC * Np, F), lambda: (0, 0)),
            pl.BlockSpec((NC * Np, 8), lambda: (0, 0)),
            pl.BlockSpec((Np, F), lambda: (0, 0)),
            pl.BlockSpec((F, F), lambda: (0, 0)),
            pl.BlockSpec((1, F), lambda: (0, 0)),
            pl.BlockSpec((F, 8), lambda: (0, 0)),
            pl.BlockSpec((1, 8), lambda: (0, 0)),
        ],
        out_specs=pl.BlockSpec((Np, 8), lambda: (0, 0)),
        out_shape=jax.ShapeDtypeStruct((Np, 8), jnp.float32),
    )(spart, cpart, h_prev, rootp, biasp, owp, obp)


def _prep_layer(w1, b1, w2, b2, root, bias, in_c, out_c):
    eye8 = jnp.eye(8, dtype=jnp.float32)
    w1c = jnp.concatenate(
        [jnp.kron(eye8, w1[:, h:h + 1] * jnp.ones((1, F), jnp.float32))
         for h in range(NH)], axis=1)
    b1c = jnp.repeat(b1, 128).reshape(1, NH * 128)
    w2r = w2.reshape(NH, in_c, out_c)
    w2r = jnp.pad(w2r, ((0, 0), (0, F - in_c), (0, F - out_c)))
    b2p = jnp.pad(b2.reshape(in_c, out_c), ((0, F - in_c), (0, F - out_c)))
    bdc = jnp.concatenate(
        [jnp.kron(eye8, w2r[h]) for h in range(NH)]
        + [jnp.kron(eye8, b2p)], axis=1)
    rootp = jnp.pad(root, ((0, F - in_c), (0, F - out_c)))
    biasp = jnp.pad(bias, (0, F - out_c)).reshape(1, F)
    return w1c, b1c, bdc, rootp, biasp


def _layer(h, ea, src1, dst1, zeros16, cpart, params, first_aux=None,
           last=False, head=None):
    w1c, b1c, bdc, rootp, biasp = params
    xj = _gather_sc(h, src1)
    msg128 = _msg_tc(ea, xj.reshape(R8, 128), w1c, b1c, bdc)
    msg = msg128.reshape(Ep, F)
    if first_aux is not None:
        zeros8, ones8 = first_aux
        spart, cpart = _scatter_cnt_sc(msg, dst1, zeros16, zeros8, ones8)
    else:
        spart = _scatter_sc(msg, dst1, zeros16)
    if last:
        owp, obp = head
        return _upd3_tc(spart, cpart, h, rootp, biasp, owp, obp)
    return _upd_tc(spart, cpart, h, rootp, biasp), cpart


def kernel(num_layers, x, edge_index, edge_attr,
           em_w1, em_b1, em_w2, em_b2,
           mn_w1, mn_b1, mn_w2, mn_b2,
           en_w1, en_b1, en_w2, en_b2,
           em_root, em_bias, l1_root, l1_bias, end_root, end_bias,
           out_w, out_b):
    ea8 = jnp.pad(edge_attr, ((0, Ep - E), (0, 0))).reshape(R8, 40)
    x_pad = jnp.pad(x, ((0, Np - N), (0, F - 10)))
    zeros16 = jnp.zeros((Np, F), jnp.float32)
    zeros8 = jnp.zeros((Np, 8), jnp.float32)
    ones8 = jnp.ones((128, 8), jnp.float32)

    p1 = _prep_layer(em_w1, em_b1, em_w2, em_b2, em_root, em_bias, 10, F)
    p2 = _prep_layer(mn_w1, mn_b1, mn_w2, mn_b2, l1_root, l1_bias, F, F)
    p3 = _prep_layer(en_w1, en_b1, en_w2, en_b2, end_root, end_bias, F, 10)
    owp = jnp.zeros((F, 8), jnp.float32).at[:10, :1].set(out_w)
    obp = jnp.zeros((1, 8), jnp.float32).at[0, :1].set(out_b)

    src1 = edge_index[0]
    dst1 = edge_index[1].reshape(E // 128, 128)
    h1, cpart = _layer(x_pad, ea8, src1, dst1, zeros16, None, p1,
                       first_aux=(zeros8, ones8))
    h_mid, _ = _layer(h1, ea8, src1, dst1, zeros16, cpart, p2)
    h2 = jnp.where(num_layers == 1, h_mid, h1)
    out = _layer(h2, ea8, src1, dst1, zeros16, cpart, p3, last=True,
                 head=(owp, obp))
    return out[:N, :1]


# speedup vs baseline: 9.0470x; 1.0035x over previous
import functools

import jax
import jax.numpy as jnp
from jax import lax
from jax.experimental import pallas as pl
from jax.experimental.pallas import tpu as pltpu
from jax.experimental.pallas import tpu_sc as plsc

N = 10000
E = 320000
Np = 10240
NC, NS = 2, 16
NW = NC * NS
C = 2048
NCHUNK = 5
EW = C * NCHUNK
Ep = EW * NW
F = 16
R8 = Ep // 8
BR = 512
NH = 10


def _mesh():
    return plsc.VectorSubcoreMesh(core_axis_name="c", subcore_axis_name="s",
                                  num_cores=NC, num_subcores=NS)


def _gather_body(h_hbm, src_hbm, xj_hbm, idx_v, rows_v, sem, wsem):
    cid = lax.axis_index("c")
    sid = lax.axis_index("s")
    wid = sid * NC + cid

    wb = {}
    for ci in range(NCHUNK):
        ebase = pl.multiple_of(wid * EW + ci * C, C)
        b = ci % 2
        if b in wb:
            wb.pop(b).wait()

        @pl.when(ebase + C <= E)
        def _full():
            pltpu.sync_copy(src_hbm.at[pl.ds(ebase, C)], idx_v)

        @pl.when(ebase + C > E)
        def _partial():
            for j in range(C // 128):
                @pl.when(ebase + (j + 1) * 128 <= E)
                def _one():
                    off = pl.multiple_of(ebase + j * 128, 128)
                    pltpu.sync_copy(src_hbm.at[pl.ds(off, 128)],
                                    idx_v.at[pl.ds(j * 128, 128)])

        descs = [
            pltpu.async_copy(h_hbm.at[idx_v.at[pl.ds(j * 128, 128)]],
                             rows_v.at[b].at[pl.ds(j * 128, 128)], sem)
            for j in range(C // 128)
        ]
        for d in descs:
            d.wait()
        wb[b] = pltpu.async_copy(rows_v.at[b], xj_hbm.at[pl.ds(ebase, C)],
                                 wsem)
    for d in wb.values():
        d.wait()


@functools.cache
def _build_gather():
    return pl.kernel(
        _gather_body,
        out_type=jax.ShapeDtypeStruct((Ep, F), jnp.float32),
        mesh=_mesh(),
        scratch_types=[
            pltpu.VMEM((C,), jnp.int32),
            pltpu.VMEM((2, C, F), jnp.float32),
            pltpu.SemaphoreType.DMA,
            pltpu.SemaphoreType.DMA,
        ],
        compiler_params=pltpu.CompilerParams(use_tc_tiling_on_sc=False),
    )


def _gather_sc(h, src1):
    return _build_gather()(h, src1)


def _load_dst_rows(dst2_hbm, idx2_v, ebase):
    rbase = pl.multiple_of(ebase // 128, C // 128)

    @pl.when(ebase + C <= E)
    def _full():
        pltpu.sync_copy(dst2_hbm.at[pl.ds(rbase, C // 128)], idx2_v)

    @pl.when(ebase + C > E)
    def _partial():
        for j in range(C // 128):
            @pl.when(ebase + (j + 1) * 128 <= E)
            def _one():
                pltpu.sync_copy(dst2_hbm.at[pl.ds(rbase + j, 1)],
                                idx2_v.at[pl.ds(j, 1)])


def _scatter_body(msg_hbm, dst_hbm, zeros_hbm, out_hbm, idx2_v, rows_v,
                  acc_sh, sem):
    cid = lax.axis_index("c")
    sid = lax.axis_index("s")
    wid = sid * NC + cid
    rpt = Np // NS

    pltpu.sync_copy(zeros_hbm.at[pl.ds(sid * rpt, rpt)],
                    acc_sh.at[pl.ds(sid * rpt, rpt)])
    plsc.subcore_barrier()

    @pl.loop(0, NCHUNK)
    def _chunk(ci):
        ebase = pl.multiple_of(wid * EW + ci * C, C)
        msg_d = pltpu.async_copy(msg_hbm.at[pl.ds(ebase, C)], rows_v, sem)
        _load_dst_rows(dst_hbm, idx2_v, ebase)
        msg_d.wait()
        for j in range(C // 128):
            @pl.when(ebase + (j + 1) * 128 <= E)
            def _add():
                pltpu.sync_copy(rows_v.at[pl.ds(j * 128, 128)],
                                acc_sh.at[idx2_v.at[j]], add=True)

    plsc.subcore_barrier()
    pltpu.sync_copy(acc_sh.at[pl.ds(sid * rpt, rpt)],
                    out_hbm.at[pl.ds(cid * Np + sid * rpt, rpt)])


@functools.cache
def _build_scatter():
    return pl.kernel(
        _scatter_body,
        out_type=jax.ShapeDtypeStruct((NC * Np, F), jnp.float32),
        mesh=_mesh(),
        scratch_types=[
            pltpu.VMEM((C // 128, 128), jnp.int32),
            pltpu.VMEM((C, F), jnp.float32),
            pltpu.VMEM_SHARED((Np, F), jnp.float32),
            pltpu.SemaphoreType.DMA,
        ],
        compiler_params=pltpu.CompilerParams(use_tc_tiling_on_sc=False),
    )


def _scatter_sc(msg, dst1, zeros16):
    return _build_scatter()(msg, dst1, zeros16)


def _scatter_cnt_body(msg_hbm, dst_hbm, zeros_hbm, zeros8_hbm, ones_hbm,
                      out_hbm, cnt_hbm, idx2_v, rows_v, ones_v, acc_sh,
                      cacc_sh, sem):
    cid = lax.axis_index("c")
    sid = lax.axis_index("s")
    wid = sid * NC + cid
    rpt = Np // NS

    pltpu.sync_copy(ones_hbm, ones_v)
    pltpu.sync_copy(zeros_hbm.at[pl.ds(sid * rpt, rpt)],
                    acc_sh.at[pl.ds(sid * rpt, rpt)])
    pltpu.sync_copy(zeros8_hbm.at[pl.ds(sid * rpt, rpt)],
                    cacc_sh.at[pl.ds(sid * rpt, rpt)])
    plsc.subcore_barrier()

    @pl.loop(0, NCHUNK)
    def _chunk(ci):
        ebase = pl.multiple_of(wid * EW + ci * C, C)
        msg_d = pltpu.async_copy(msg_hbm.at[pl.ds(ebase, C)], rows_v, sem)
        _load_dst_rows(dst_hbm, idx2_v, ebase)
        msg_d.wait()
        for j in range(C // 128):
            @pl.when(ebase + (j + 1) * 128 <= E)
            def _add():
                pltpu.sync_copy(rows_v.at[pl.ds(j * 128, 128)],
                                acc_sh.at[idx2_v.at[j]], add=True)
                pltpu.sync_copy(ones_v, cacc_sh.at[idx2_v.at[j]], add=True)

    plsc.subcore_barrier()
    pltpu.sync_copy(acc_sh.at[pl.ds(sid * rpt, rpt)],
                    out_hbm.at[pl.ds(cid * Np + sid * rpt, rpt)])
    pltpu.sync_copy(cacc_sh.at[pl.ds(sid * rpt, rpt)],
                    cnt_hbm.at[pl.ds(cid * Np + sid * rpt, rpt)])


@functools.cache
def _build_scatter_cnt():
    return pl.kernel(
        _scatter_cnt_body,
        out_type=[jax.ShapeDtypeStruct((NC * Np, F), jnp.float32),
                  jax.ShapeDtypeStruct((NC * Np, 8), jnp.float32)],
        mesh=_mesh(),
        scratch_types=[
            pltpu.VMEM((C // 128, 128), jnp.int32),
            pltpu.VMEM((C, F), jnp.float32),
            pltpu.VMEM((128, 8), jnp.float32),
            pltpu.VMEM_SHARED((Np, F), jnp.float32),
            pltpu.VMEM_SHARED((Np, 8), jnp.float32),
            pltpu.SemaphoreType.DMA,
        ],
        compiler_params=pltpu.CompilerParams(use_tc_tiling_on_sc=False),
    )


def _scatter_cnt_sc(msg, dst1, zeros16, zeros8, ones8):
    return _build_scatter_cnt()(msg, dst1, zeros16, zeros8, ones8)


def _msg_body(ea_ref, xj_ref, w1c_ref, b1c_ref, bdc_ref, out_ref):
    z = ea_ref[...]
    xjb = xj_ref[...]
    a = jnp.maximum(
        jnp.dot(z, w1c_ref[...],
                preferred_element_type=jnp.float32) + b1c_ref[...], 0.0)
    y = jnp.dot(xjb, bdc_ref[...], preferred_element_type=jnp.float32)
    acc = y[:, NH * 128:(NH + 1) * 128]
    for h in range(NH):
        acc = acc + a[:, h * 128:(h + 1) * 128] * y[:, h * 128:(h + 1) * 128]
    out_ref[...] = acc


def _msg_tc(ea, xj128, w1c, b1c, bdc):
    return pl.pallas_call(
        _msg_body,
        grid=(R8 // BR,),
        in_specs=[
            pl.BlockSpec((BR, 40), lambda i: (i, 0)),
            pl.BlockSpec((BR, 128), lambda i: (i, 0)),
            pl.BlockSpec((40, NH * 128), lambda i: (0, 0)),
            pl.BlockSpec((1, NH * 128), lambda i: (0, 0)),
            pl.BlockSpec((128, (NH + 1) * 128), lambda i: (0, 0)),
        ],
        out_specs=pl.BlockSpec((BR, 128), lambda i: (i, 0)),
        out_shape=jax.ShapeDtypeStruct((R8, 128), jnp.float32),
    )(ea, xj128, w1c, b1c, bdc)


def _upd_body(sp_ref, cp_ref, h_ref, root_ref, bias_ref, out_ref):
    s = sp_ref[pl.ds(0, Np), :] + sp_ref[pl.ds(Np, Np), :]
    cnt = cp_ref[pl.ds(0, Np), :] + cp_ref[pl.ds(Np, Np), :]
    mean = s / jnp.maximum(cnt[:, 0:1], 1.0)
    out_ref[...] = jnp.maximum(
        mean + jnp.dot(h_ref[...], root_ref[...],
                       preferred_element_type=jnp.float32) + bias_ref[...],
        0.0)


def _upd_tc(spart, cpart, h_prev, rootp, biasp):
    return pl.pallas_call(
        _upd_body,
        in_specs=[
            pl.BlockSpec((NC * Np, F), lambda: (0, 0)),
            pl.BlockSpec((NC * Np, 8), lambda: (0, 0)),
            pl.BlockSpec((Np, F), lambda: (0, 0)),
            pl.BlockSpec((F, F), lambda: (0, 0)),
            pl.BlockSpec((1, F), lambda: (0, 0)),
        ],
        out_specs=pl.BlockSpec((Np, F), lambda: (0, 0)),
        out_shape=jax.ShapeDtypeStruct((Np, F), jnp.float32),
    )(spart, cpart, h_prev, rootp, biasp)


def _upd3_body(sp_ref, cp_ref, h_ref, root_ref, bias_ref, ow_ref, ob_ref,
               out_ref):
    s = sp_ref[pl.ds(0, Np), :] + sp_ref[pl.ds(Np, Np), :]
    cnt = cp_ref[pl.ds(0, Np), :] + cp_ref[pl.ds(Np, Np), :]
    mean = s / jnp.maximum(cnt[:, 0:1], 1.0)
    h3 = jnp.maximum(
        mean + jnp.dot(h_ref[...], root_ref[...],
                       preferred_element_type=jnp.float32) + bias_ref[...],
        0.0)
    out_ref[...] = jnp.dot(h3, ow_ref[...],
                           preferred_element_type=jnp.float32) + ob_ref[...]


def _upd3_tc(spart, cpart, h_prev, rootp, biasp, owp, obp):
    return pl.pallas_call(
        _upd3_body,
        in_specs=[
            pl.BlockSpec((NC * Np, F), lambda: (0, 0)),
            pl.BlockSpec((NC * Np, 8), lambda: (0, 0)),
            pl.BlockSpec((Np, F), lambda: (0, 0)),
            pl.BlockSpec((F, F), lambda: (0, 0)),
            pl.BlockSpec((1, F), lambda: (0, 0)),
            pl.BlockSpec((F, 8), lambda: (0, 0)),
            pl.BlockSpec((1, 8), lambda: (0, 0)),
        ],
        out_specs=pl.BlockSpec((Np, 8), lambda: (0, 0)),
        out_shape=jax.ShapeDtypeStruct((Np, 8), jnp.float32),
    )(spart, cpart, h_prev, rootp, biasp, owp, obp)


def _prep_layer(w1, b1, w2, b2, root, bias, in_c, out_c):
    eye8 = jnp.eye(8, dtype=jnp.float32)
    w1c = jnp.concatenate(
        [jnp.kron(eye8, w1[:, h:h + 1] * jnp.ones((1, F), jnp.float32))
         for h in range(NH)], axis=1)
    b1c = jnp.repeat(b1, 128).reshape(1, NH * 128)
    w2r = w2.reshape(NH, in_c, out_c)
    w2r = jnp.pad(w2r, ((0, 0), (0, F - in_c), (0, F - out_c)))
    b2p = jnp.pad(b2.reshape(in_c, out_c), ((0, F - in_c), (0, F - out_c)))
    bdc = jnp.concatenate(
        [jnp.kron(eye8, w2r[h]) for h in range(NH)]
        + [jnp.kron(eye8, b2p)], axis=1)
    rootp = jnp.pad(root, ((0, F - in_c), (0, F - out_c)))
    biasp = jnp.pad(bias, (0, F - out_c)).reshape(1, F)
    return w1c, b1c, bdc, rootp, biasp


def _layer(h, ea, src1, dst1, zeros16, cpart, params, first_aux=None,
           last=False, head=None):
    w1c, b1c, bdc, rootp, biasp = params
    xj = _gather_sc(h, src1)
    msg128 = _msg_tc(ea, xj.reshape(R8, 128), w1c, b1c, bdc)
    msg = msg128.reshape(Ep, F)
    if first_aux is not None:
        zeros8, ones8 = first_aux
        spart, cpart = _scatter_cnt_sc(msg, dst1, zeros16, zeros8, ones8)
    else:
        spart = _scatter_sc(msg, dst1, zeros16)
    if last:
        owp, obp = head
        return _upd3_tc(spart, cpart, h, rootp, biasp, owp, obp)
    return _upd_tc(spart, cpart, h, rootp, biasp), cpart


def kernel(num_layers, x, edge_index, edge_attr,
           em_w1, em_b1, em_w2, em_b2,
           mn_w1, mn_b1, mn_w2, mn_b2,
           en_w1, en_b1, en_w2, en_b2,
           em_root, em_bias, l1_root, l1_bias, end_root, end_bias,
           out_w, out_b):
    ea8 = jnp.pad(edge_attr, ((0, Ep - E), (0, 0))).reshape(R8, 40)
    x_pad = jnp.pad(x, ((0, Np - N), (0, F - 10)))
    zeros16 = jnp.zeros((Np, F), jnp.float32)
    zeros8 = jnp.zeros((Np, 8), jnp.float32)
    ones8 = jnp.ones((128, 8), jnp.float32)

    p1 = _prep_layer(em_w1, em_b1, em_w2, em_b2, em_root, em_bias, 10, F)
    p2 = _prep_layer(mn_w1, mn_b1, mn_w2, mn_b2, l1_root, l1_bias, F, F)
    p3 = _prep_layer(en_w1, en_b1, en_w2, en_b2, end_root, end_bias, F, 10)
    owp = jnp.zeros((F, 8), jnp.float32).at[:10, :1].set(out_w)
    obp = jnp.zeros((1, 8), jnp.float32).at[0, :1].set(out_b)

    src1 = edge_index[0]
    dst1 = edge_index[1].reshape(E // 128, 128)
    h1, cpart = _layer(x_pad, ea8, src1, dst1, zeros16, None, p1,
                       first_aux=(zeros8, ones8))
    h_mid, _ = _layer(h1, ea8, src1, dst1, zeros16, cpart, p2)
    h2 = jnp.where(num_layers == 1, h_mid, h1)
    out = _layer(h2, ea8, src1, dst1, zeros16, cpart, p3, last=True,
                 head=(owp, obp))
    return out[:N, :1]
